# baseline (speedup 1.0000x reference)
import functools
import math

import jax
import jax.numpy as jnp
from jax import lax
from jax.experimental import pallas as pl
from jax.experimental.pallas import tpu as pltpu
from jax.experimental.pallas import tpu_sc as plsc

DIM_ = 192
HEADS_ = 4
TOPK_START_ = 0.05
TOPK_END_ = 0.15
ALPHA_MAX_ = 0.35
GAMMA_ = 0.5
WARMUP_ = 1500
STEP_ = 1
EPS_ = 1e-6
BPP_ = 8
LANES_ = 16


def _stage_a_body(f_ref, qk_s_ref, lconst_ref, lcls_ref, xs_cls_ref,
                  post_tok_ref, v_wt_ref, v_b_ref, o_wt_ref, o_b_ref,
                  pre_w_ref, pre_b_ref, asp_ref, sal_ref, a1_ref, b1_ref,
                  vpre_ref):
    C = DIM_
    T = f_ref.shape[1]
    dh = C // HEADS_
    bf = jnp.bfloat16

    pre_w = pre_w_ref[...]
    pre_b = pre_b_ref[...]
    xs_cls = xs_cls_ref[...]
    lcls = lcls_ref[...].reshape(HEADS_, 1)
    sel4 = (lax.broadcasted_iota(jnp.int32, (HEADS_, C), 1) // dh
            == lax.broadcasted_iota(jnp.int32, (HEADS_, C), 0))
    ones_row = jnp.ones((1, T), bf)

    sal_cols = []
    for i in range(BPP_):
        ft = f_ref[i]

        u = jnp.mean(ft, axis=0, keepdims=True)
        msq = jnp.mean(ft * ft, axis=0, keepdims=True)
        inv = lax.rsqrt(msq - u * u + EPS_)
        a1 = pre_w * inv
        b1 = pre_b - u * a1
        a1_ref[i] = a1
        b1_ref[i] = b1
        x = ft * a1 + b1

        logits = lax.dot_general(qk_s_ref[...], x, (((1,), (1,)), ((), ())),
                                 preferred_element_type=jnp.float32)
        logits = logits + lconst_ref[...]
        m = jnp.maximum(jnp.max(logits, axis=1, keepdims=True), lcls)
        e = jnp.exp(logits - m)
        e_cls = jnp.exp(lcls - m)
        z = jnp.sum(e, axis=1, keepdims=True) + e_cls
        w_attn = e / z
        w_cls = e_cls / z

        asp = jnp.sum(w_attn, axis=0, keepdims=True) * (1.0 / HEADS_)
        asp_ref[i] = asp / (jnp.max(asp) + 1e-6)

        x_bf = x.astype(bf)
        w_bf = w_attn.astype(bf)
        m5 = jnp.concatenate([w_bf, ones_row], axis=0)
        r5 = jnp.dot(m5, x_bf, preferred_element_type=jnp.float32)
        s = r5[:HEADS_] + jnp.dot(w_bf, post_tok_ref[...],
                                  preferred_element_type=jnp.float32)
        v_glb = r5[HEADS_:] * (1.0 / T)

        z_heads = s + w_cls * xs_cls
        v4 = jnp.dot(z_heads.astype(bf), v_wt_ref[...],
                     preferred_element_type=jnp.float32)
        pooled = jnp.sum(jnp.where(sel4, v4, 0.0), axis=0, keepdims=True)
        pooled = pooled + v_b_ref[...]
        pooled = jnp.dot(pooled.astype(bf), o_wt_ref[...],
                         preferred_element_type=jnp.float32) + o_b_ref[...]

        vpre_ref[i] = GAMMA_ * v_glb + (1.0 - GAMMA_) * 0.8 * pooled

        x2 = x_bf * x_bf
        sal_cols.append(jnp.dot(x2, jnp.ones((C, 1), bf),
                                preferred_element_type=jnp.float32))

    sal_ref[...] = jnp.transpose(jnp.concatenate(sal_cols, axis=1))


def _sc_topk(sal, kc):
    B, T = sal.shape
    nchunk = T // LANES_
    mesh = plsc.VectorSubcoreMesh(core_axis_name="c", subcore_axis_name="s")

    @functools.partial(
        pl.kernel,
        mesh=mesh,
        out_type=jax.ShapeDtypeStruct((B, T), jnp.float32),
        scratch_types=[
            pltpu.VMEM((T,), jnp.float32),
            pltpu.VMEM((T,), jnp.float32),
        ],
    )
    def k(sal_hbm, wsel_hbm, sal_v, w_v):
        wid = lax.axis_index("s") * 2 + lax.axis_index("c")
        pltpu.sync_copy(sal_hbm.at[wid], sal_v)

        iot = lax.iota(jnp.int32, LANES_)

        def lane_all(vec, op):
            for sh in (1, 2, 4, 8):
                vec = op(vec, jnp.take(vec, jnp.bitwise_xor(iot, sh)))
            return vec

        def count_ge(thr):
            acc = jnp.zeros((LANES_,), jnp.int32)
            for j in range(nchunk):
                v = sal_v[pl.ds(j * LANES_, LANES_)]
                acc = acc + jnp.where(v >= thr, 1, 0)
            return lane_all(acc, jnp.add)

        mx = jnp.zeros((LANES_,), jnp.float32)
        for j in range(nchunk):
            mx = jnp.maximum(mx, sal_v[pl.ds(j * LANES_, LANES_)])
        hi0 = lane_all(mx, jnp.maximum)

        def bs_body(_, carry):
            lo, hi = carry
            mid = 0.5 * (lo + hi)
            big = count_ge(mid) >= kc
            return jnp.where(big, mid, lo), jnp.where(big, hi, mid)

        tau, _ = lax.fori_loop(0, 40, bs_body,
                               (jnp.zeros((LANES_,), jnp.float32), hi0))

        c_ge = count_ge(tau)
        acc = jnp.zeros((LANES_,), jnp.int32)
        for j in range(nchunk):
            v = sal_v[pl.ds(j * LANES_, LANES_)]
            acc = acc + jnp.where(v > tau, 1, 0)
        c_gt = lane_all(acc, jnp.add)
        n_eq = jnp.maximum(c_ge - c_gt, 1)
        w_eq = (kc - c_gt).astype(jnp.float32) / n_eq.astype(jnp.float32)
        one = jnp.ones((LANES_,), jnp.float32)
        zero = jnp.zeros((LANES_,), jnp.float32)
        for j in range(nchunk):
            v = sal_v[pl.ds(j * LANES_, LANES_)]
            w = jnp.where(v > tau, one, jnp.where(v == tau, w_eq, zero))
            w_v[pl.ds(j * LANES_, LANES_)] = w
        pltpu.sync_copy(w_v, wsel_hbm.at[wid])

    return k(sal)


def _stage_b_body(f_ref, a1_ref, b1_ref, asp_ref, vpre_ref, wsel_ref,
                  post_w_ref, post_b_ref, g1_wt_ref, g2_wt_ref, g2_b_ref,
                  out_ref, *, kc, alpha):
    C = DIM_
    T = f_ref.shape[1]
    bf = jnp.bfloat16
    post_w = post_w_ref[...]
    post_b = post_b_ref[...]
    ones_row = jnp.ones((1, T), bf)

    for i in range(BPP_):
        ft = f_ref[i]
        x_bf = (ft * a1_ref[i] + b1_ref[i]).astype(bf)

        wrow = wsel_ref[i:i + 1].astype(bf)
        refine = jnp.dot(wrow, x_bf,
                         preferred_element_type=jnp.float32) * (1.0 / kc)
        v_fused = vpre_ref[i] + (1.0 - GAMMA_) * 0.2 * refine

        h1 = jnp.dot(v_fused.astype(bf), g1_wt_ref[...],
                     preferred_element_type=jnp.float32)
        h1 = jnp.maximum(h1, 0.0)
        g = jnp.dot(h1.astype(bf), g2_wt_ref[...],
                    preferred_element_type=jnp.float32) + g2_b_ref[...]
        gate = (1.0 / (1.0 + jnp.exp(-g))).astype(bf)

        asp1 = jnp.transpose(1.0 + asp_ref[i]).astype(bf)
        fused = (x_bf * asp1) * gate

        sum2 = jnp.dot(ones_row, fused, preferred_element_type=jnp.float32)
        msq2 = jnp.dot(ones_row, fused * fused,
                       preferred_element_type=jnp.float32)
        u2 = sum2 * (1.0 / T)
        var2 = msq2 * (1.0 / T) - u2 * u2
        inv2 = lax.rsqrt(var2 + EPS_)
        a2 = alpha * (post_w * inv2)
        b2 = alpha * post_b - u2 * a2
        out_ref[i] = (ft * (1.0 - alpha)
                      + fused.astype(jnp.float32) * a2 + b2)


def kernel(feat_2d, pos, q_w, q_b, k_w, k_b, v_w, v_b, o_w, o_b,
           pre_w, pre_b, post_w, post_b, g1_w, g2_w, g2_b):
    B, C, H, W = feat_2d.shape
    T = H * W
    dh = C // HEADS_

    t = float(min(STEP_, WARMUP_))
    ratio = 0.5 * (1.0 - math.cos(math.pi * t / WARMUP_))
    alpha = ratio * ALPHA_MAX_
    topk_ratio = TOPK_START_ + (TOPK_END_ - TOPK_START_) * ratio
    kc = max(1, int(T * topk_ratio))

    ftok = feat_2d.transpose(0, 2, 3, 1).reshape(B, T, C)

    pos0 = pos[:1, :]
    post_tok = pos[1:, :]

    xs_cls = pre_b.reshape(1, C) + pos0
    q_vec = xs_cls @ q_w.T + q_b.reshape(1, C)
    head_mask = (jnp.arange(C)[None, :] // dh) == jnp.arange(HEADS_)[:, None]
    q4 = jnp.where(head_mask, q_vec, 0.0)
    inv_sqrt_dh = 1.0 / math.sqrt(dh)
    qk_s = (q4 @ k_w) * inv_sqrt_dh
    kb_term = (q4 @ k_b.reshape(C, 1)) * inv_sqrt_dh
    lconst = qk_s @ post_tok.T + kb_term
    lcls = (qk_s @ xs_cls.T + kb_term).reshape(1, HEADS_)

    bfc = lambda a: a.astype(jnp.bfloat16)
    row = lambda v: v.reshape(1, C)
    full = lambda shape: pl.BlockSpec(shape, lambda b: (0,) * len(shape))
    bspec = lambda *shape: pl.BlockSpec(shape, lambda b: (b,) + (0,) * (len(shape) - 1))

    asp, salr, a1, b1, vpre = pl.pallas_call(
        _stage_a_body,
        grid=(B // BPP_,),
        in_specs=[
            bspec(BPP_, T, C),
            full((HEADS_, C)),
            full((HEADS_, T)),
            full((1, HEADS_)),
            full((1, C)),
            full((T, C)),
            full((C, C)),
            full((1, C)),
            full((C, C)),
            full((1, C)),
            full((1, C)),
            full((1, C)),
        ],
        out_specs=[
            bspec(BPP_, 1, T),
            bspec(BPP_, T),
            bspec(BPP_, 1, C),
            bspec(BPP_, 1, C),
            bspec(BPP_, 1, C),
        ],
        out_shape=[
            jax.ShapeDtypeStruct((B, 1, T), jnp.float32),
            jax.ShapeDtypeStruct((B, T), jnp.float32),
            jax.ShapeDtypeStruct((B, 1, C), jnp.float32),
            jax.ShapeDtypeStruct((B, 1, C), jnp.float32),
            jax.ShapeDtypeStruct((B, 1, C), jnp.float32),
        ],
    )(ftok, qk_s, lconst, lcls, xs_cls, bfc(post_tok), bfc(v_w.T), row(v_b),
      bfc(o_w.T), row(o_b), row(pre_w), row(pre_b))

    wsel = _sc_topk(salr, kc)

    body_b = functools.partial(_stage_b_body, kc=kc, alpha=alpha)
    out = pl.pallas_call(
        body_b,
        grid=(B // BPP_,),
        in_specs=[
            bspec(BPP_, T, C),
            bspec(BPP_, 1, C),
            bspec(BPP_, 1, C),
            bspec(BPP_, 1, T),
            bspec(BPP_, 1, C),
            bspec(BPP_, T),
            full((1, C)),
            full((1, C)),
            full((C, C // 4)),
            full((C // 4, C)),
            full((1, C)),
        ],
        out_specs=pl.BlockSpec((BPP_, T, C), lambda b: (b, 0, 0)),
        out_shape=jax.ShapeDtypeStruct((B, T, C), jnp.float32),
    )(ftok, a1, b1, asp, vpre, wsel, row(post_w), row(post_b),
      bfc(g1_w.T), bfc(g2_w.T), row(g2_b))

    out4 = out.reshape(B, H, W, C).transpose(0, 3, 1, 2)
    return out4, asp.reshape(B, 1, H, W)
